# baseline (speedup 1.0000x reference)
import functools

import jax
import jax.numpy as jnp
from jax import lax
from jax.experimental import pallas as pl
from jax.experimental.pallas import tpu as pltpu
from jax.experimental.pallas import tpu_sc as plsc

VOCAB = 1000000
EMB = 64
FVOCAB = 100000
FEMB = 16
B = 4096
L = 50

NC = 2
NS = 16
NW = NC * NS
BT = B // NW
OUT_D = EMB + 2 * FEMB
CT = OUT_D // 8


def _sc_gather_concat():
    mesh = plsc.VectorSubcoreMesh(core_axis_name="c", subcore_axis_name="s")

    @functools.partial(
        pl.kernel,
        out_type=jax.ShapeDtypeStruct((L, CT, NW, 8, BT), jnp.float32),
        mesh=mesh,
        compiler_params=pltpu.CompilerParams(use_tc_tiling_on_sc=False, needs_layout_passes=False),
        scratch_types=[
            pltpu.VMEM((L, BT), jnp.int32),
            pltpu.VMEM((L, BT), jnp.int32),
            pltpu.VMEM((L, BT), jnp.int32),
            pltpu.VMEM((2, BT, EMB), jnp.float32),
            pltpu.VMEM((2, BT, FEMB), jnp.float32),
            pltpu.VMEM((2, BT, FEMB), jnp.float32),
            pltpu.VMEM((CT, 8, BT), jnp.float32),
            pltpu.SemaphoreType.DMA,
            pltpu.SemaphoreType.DMA,
            pltpu.SemaphoreType.DMA,
            pltpu.SemaphoreType.DMA,
        ],
    )
    def k(widx_hbm, f0idx_hbm, f1idx_hbm, wtab_hbm, f0tab_hbm, f1tab_hbm,
          out_hbm, widx_v, f0idx_v, f1idx_v, wrows, f0rows, f1rows, cat,
          sem_gw, sem_g0, sem_g1, sem_o):
        wid = lax.axis_index("s") * NC + lax.axis_index("c")
        b0 = wid * BT
        pltpu.sync_copy(widx_hbm.at[:, pl.ds(b0, BT)], widx_v)
        pltpu.sync_copy(f0idx_hbm.at[:, pl.ds(b0, BT)], f0idx_v)
        pltpu.sync_copy(f1idx_hbm.at[:, pl.ds(b0, BT)], f1idx_v)

        lane = lax.iota(jnp.int32, 16)

        def gathers(l, s):
            pltpu.async_copy(wtab_hbm.at[widx_v.at[l]], wrows.at[s], sem_gw)
            pltpu.async_copy(f0tab_hbm.at[f0idx_v.at[l]], f0rows.at[s], sem_g0)
            pltpu.async_copy(f1tab_hbm.at[f1idx_v.at[l]], f1rows.at[s], sem_g1)

        def wait_gathers(l, s):
            pltpu.make_async_copy(wtab_hbm.at[widx_v.at[l]], wrows.at[s], sem_gw).wait()
            pltpu.make_async_copy(f0tab_hbm.at[f0idx_v.at[l]], f0rows.at[s], sem_g0).wait()
            pltpu.make_async_copy(f1tab_hbm.at[f1idx_v.at[l]], f1rows.at[s], sem_g1).wait()

        def transpose_into_cat(s):
            slot = lane * 0 + s

            def tw(ct, carry):
                for cr in range(8):
                    col = lane * 0 + (ct * 8 + cr)
                    for t0 in range(0, BT, 16):
                        v = plsc.load_gather(wrows, [slot, lane + t0, col])
                        cat[ct, cr, pl.ds(t0, 16)] = v
                return carry

            lax.fori_loop(0, EMB // 8, tw, 0)

            def tf(ct2, carry, rows, ctbase):
                for cr in range(8):
                    col = lane * 0 + (ct2 * 8 + cr)
                    for t0 in range(0, BT, 16):
                        v = plsc.load_gather(rows, [slot, lane + t0, col])
                        cat[ctbase + ct2, cr, pl.ds(t0, 16)] = v
                return carry

            lax.fori_loop(0, FEMB // 8, functools.partial(
                tf, rows=f0rows, ctbase=EMB // 8), 0)
            lax.fori_loop(0, FEMB // 8, functools.partial(
                tf, rows=f1rows, ctbase=(EMB + FEMB) // 8), 0)

        def out_slice(l):
            return out_hbm.at[l, :, wid, :, :]

        gathers(0, 0)

        def step(g, carry):
            l0 = g * 2
            pl.when(l0 + 1 < L)(lambda: gathers(l0 + 1, 1))
            wait_gathers(l0, 0)
            transpose_into_cat(0)
            pltpu.sync_copy(cat, out_slice(l0))
            pl.when(l0 + 2 < L)(lambda: gathers(l0 + 2, 0))
            wait_gathers(l0 + 1, 1)
            transpose_into_cat(1)
            pltpu.sync_copy(cat, out_slice(l0 + 1))
            return carry

        lax.fori_loop(0, L // 2, step, 0)

    return k


_GATHER = _sc_gather_concat()


def kernel(word_inputs, feature_inputs_0, feature_inputs_1, word_seq_lengths,
           char_inputs, char_seq_lengths, char_seq_recover,
           word_table, feat_table_0, feat_table_1):
    out5 = _GATHER(word_inputs.T.astype(jnp.int32),
                   feature_inputs_0.T.astype(jnp.int32),
                   feature_inputs_1.T.astype(jnp.int32),
                   word_table, feat_table_0, feat_table_1)
    return jnp.transpose(out5, (2, 4, 0, 1, 3)).reshape(B, L, OUT_D)


# speedup vs baseline: 1.0385x; 1.0385x over previous
import functools

import jax
import jax.numpy as jnp
from jax import lax
from jax.experimental import pallas as pl
from jax.experimental.pallas import tpu as pltpu
from jax.experimental.pallas import tpu_sc as plsc

VOCAB = 1000000
EMB = 64
FVOCAB = 100000
FEMB = 16
B = 4096
L = 50

NC = 2
NS = 16
NW = NC * NS
BT = B // NW
OUT_D = EMB + 2 * FEMB
CT = OUT_D // 8


def _sc_gather_concat():
    mesh = plsc.VectorSubcoreMesh(core_axis_name="c", subcore_axis_name="s")

    @functools.partial(
        pl.kernel,
        out_type=jax.ShapeDtypeStruct((L, CT, NW, 8 * BT), jnp.float32),
        mesh=mesh,
        compiler_params=pltpu.CompilerParams(
            use_tc_tiling_on_sc=False, needs_layout_passes=False),
        scratch_types=[
            pltpu.VMEM((L, BT), jnp.int32),
            pltpu.VMEM((L, BT), jnp.int32),
            pltpu.VMEM((L, BT), jnp.int32),
            pltpu.VMEM((2, BT, EMB), jnp.float32),
            pltpu.VMEM((2, BT, FEMB), jnp.float32),
            pltpu.VMEM((2, BT, FEMB), jnp.float32),
            pltpu.VMEM((CT, 8 * BT), jnp.float32),
            pltpu.SemaphoreType.DMA,
            pltpu.SemaphoreType.DMA,
            pltpu.SemaphoreType.DMA,
        ],
    )
    def k(widx_hbm, f0idx_hbm, f1idx_hbm, wtab_hbm, f0tab_hbm, f1tab_hbm,
          out_hbm, widx_v, f0idx_v, f1idx_v, wrows, f0rows, f1rows, cat,
          sem_gw, sem_g0, sem_g1):
        wid = lax.axis_index("s") * NC + lax.axis_index("c")
        b0 = wid * BT
        pltpu.sync_copy(widx_hbm.at[:, pl.ds(b0, BT)], widx_v)
        pltpu.sync_copy(f0idx_hbm.at[:, pl.ds(b0, BT)], f0idx_v)
        pltpu.sync_copy(f1idx_hbm.at[:, pl.ds(b0, BT)], f1idx_v)

        lane = lax.iota(jnp.int32, 16)
        groups = []
        for c0 in range(0, OUT_D, 16):
            cvec = lane + c0
            groups.append((lax.shift_right_logical(cvec, 3),
                           lax.shift_left(lax.bitwise_and(cvec, 7), 7)))

        def gathers(l, s):
            pltpu.async_copy(wtab_hbm.at[widx_v.at[l]], wrows.at[s], sem_gw)
            pltpu.async_copy(f0tab_hbm.at[f0idx_v.at[l]], f0rows.at[s], sem_g0)
            pltpu.async_copy(f1tab_hbm.at[f1idx_v.at[l]], f1rows.at[s], sem_g1)

        def wait_gathers(l, s):
            pltpu.make_async_copy(wtab_hbm.at[widx_v.at[l]], wrows.at[s], sem_gw).wait()
            pltpu.make_async_copy(f0tab_hbm.at[f0idx_v.at[l]], f0rows.at[s], sem_g0).wait()
            pltpu.make_async_copy(f1tab_hbm.at[f1idx_v.at[l]], f1rows.at[s], sem_g1).wait()

        def transpose_into_cat(s):
            def tt(t, carry):
                for i in range(EMB // 16):
                    ia, colb = groups[i]
                    v = wrows[s, t, pl.ds(i * 16, 16)]
                    plsc.store_scatter(cat, [ia, colb + t], v)
                ia, colb = groups[EMB // 16]
                plsc.store_scatter(cat, [ia, colb + t], f0rows[s, t, pl.ds(0, FEMB)])
                ia, colb = groups[EMB // 16 + 1]
                plsc.store_scatter(cat, [ia, colb + t], f1rows[s, t, pl.ds(0, FEMB)])
                return carry

            lax.fori_loop(0, BT, tt, 0)

        def out_slice(l):
            return out_hbm.at[l, :, wid, :]

        gathers(0, 0)

        def step(g, carry):
            l0 = g * 2
            pl.when(l0 + 1 < L)(lambda: gathers(l0 + 1, 1))
            wait_gathers(l0, 0)
            transpose_into_cat(0)
            pltpu.sync_copy(cat, out_slice(l0))
            pl.when(l0 + 2 < L)(lambda: gathers(l0 + 2, 0))
            wait_gathers(l0 + 1, 1)
            transpose_into_cat(1)
            pltpu.sync_copy(cat, out_slice(l0 + 1))
            return carry

        lax.fori_loop(0, L // 2, step, 0)

    return k


_GATHER = _sc_gather_concat()


def kernel(word_inputs, feature_inputs_0, feature_inputs_1, word_seq_lengths,
           char_inputs, char_seq_lengths, char_seq_recover,
           word_table, feat_table_0, feat_table_1):
    out4 = _GATHER(word_inputs.T.astype(jnp.int32),
                   feature_inputs_0.T.astype(jnp.int32),
                   feature_inputs_1.T.astype(jnp.int32),
                   word_table, feat_table_0, feat_table_1)
    out5 = out4.reshape(L, CT, NW, 8, BT)
    return jnp.transpose(out5, (2, 4, 0, 1, 3)).reshape(B, L, OUT_D)


# speedup vs baseline: 1.0633x; 1.0239x over previous
import functools

import jax
import jax.numpy as jnp
from jax import lax
from jax.experimental import pallas as pl
from jax.experimental.pallas import tpu as pltpu
from jax.experimental.pallas import tpu_sc as plsc

VOCAB = 1000000
EMB = 64
FVOCAB = 100000
FEMB = 16
B = 4096
L = 50

NC = 2
NS = 16
NW = NC * NS
BT = B // NW
OUT_D = EMB + 2 * FEMB
CT = OUT_D // 8


def _sc_gather_concat():
    mesh = plsc.VectorSubcoreMesh(core_axis_name="c", subcore_axis_name="s")

    @functools.partial(
        pl.kernel,
        out_type=jax.ShapeDtypeStruct((L, CT, NW, 8 * BT), jnp.float32),
        mesh=mesh,
        compiler_params=pltpu.CompilerParams(
            use_tc_tiling_on_sc=False, needs_layout_passes=False),
        scratch_types=[
            pltpu.VMEM((L, BT), jnp.int32),
            pltpu.VMEM((L, BT), jnp.int32),
            pltpu.VMEM((L, BT), jnp.int32),
            pltpu.VMEM((2, BT, EMB), jnp.float32),
            pltpu.VMEM((2, BT, FEMB), jnp.float32),
            pltpu.VMEM((2, BT, FEMB), jnp.float32),
            pltpu.VMEM((2, CT, 8 * BT), jnp.float32),
            pltpu.SemaphoreType.DMA,
            pltpu.SemaphoreType.DMA,
            pltpu.SemaphoreType.DMA,
            pltpu.SemaphoreType.DMA,
            pltpu.SemaphoreType.DMA,
        ],
    )
    def k(widx_hbm, f0idx_hbm, f1idx_hbm, wtab_hbm, f0tab_hbm, f1tab_hbm,
          out_hbm, widx_v, f0idx_v, f1idx_v, wrows, f0rows, f1rows, cat,
          sem_gw, sem_g0, sem_g1, sem_o0, sem_o1):
        wid = lax.axis_index("s") * NC + lax.axis_index("c")
        b0 = wid * BT
        pltpu.sync_copy(widx_hbm.at[:, pl.ds(b0, BT)], widx_v)
        pltpu.sync_copy(f0idx_hbm.at[:, pl.ds(b0, BT)], f0idx_v)
        pltpu.sync_copy(f1idx_hbm.at[:, pl.ds(b0, BT)], f1idx_v)

        lane = lax.iota(jnp.int32, 16)
        groups = []
        for c0 in range(0, OUT_D, 16):
            cvec = lane + c0
            groups.append((lax.shift_right_logical(cvec, 3),
                           lax.shift_left(lax.bitwise_and(cvec, 7), 7)))

        def gathers(l, s):
            pltpu.async_copy(wtab_hbm.at[widx_v.at[l]], wrows.at[s], sem_gw)
            pltpu.async_copy(f0tab_hbm.at[f0idx_v.at[l]], f0rows.at[s], sem_g0)
            pltpu.async_copy(f1tab_hbm.at[f1idx_v.at[l]], f1rows.at[s], sem_g1)

        def wait_gathers(l, s):
            pltpu.make_async_copy(wtab_hbm.at[widx_v.at[l]], wrows.at[s], sem_gw).wait()
            pltpu.make_async_copy(f0tab_hbm.at[f0idx_v.at[l]], f0rows.at[s], sem_g0).wait()
            pltpu.make_async_copy(f1tab_hbm.at[f1idx_v.at[l]], f1rows.at[s], sem_g1).wait()

        def transpose_into_cat(s):
            cs = cat.at[s]

            def tt(tq, carry):
                for u in range(4):
                    t = tq * 4 + u
                    for i in range(EMB // 16):
                        ia, colb = groups[i]
                        v = wrows[s, t, pl.ds(i * 16, 16)]
                        plsc.store_scatter(cs, [ia, colb + t], v)
                    ia, colb = groups[EMB // 16]
                    plsc.store_scatter(cs, [ia, colb + t],
                                       f0rows[s, t, pl.ds(0, FEMB)])
                    ia, colb = groups[EMB // 16 + 1]
                    plsc.store_scatter(cs, [ia, colb + t],
                                       f1rows[s, t, pl.ds(0, FEMB)])
                return carry

            lax.fori_loop(0, BT // 4, tt, 0)

        def out_slice(l):
            return out_hbm.at[l, :, wid, :]

        def wait_write(l, s, sem):
            pltpu.make_async_copy(cat.at[s], out_slice(l), sem).wait()

        gathers(0, 0)

        def step(g, carry):
            l0 = g * 2
            pl.when(l0 + 1 < L)(lambda: gathers(l0 + 1, 1))
            wait_gathers(l0, 0)
            pl.when(g > 0)(lambda: wait_write(l0, 0, sem_o0))
            transpose_into_cat(0)
            pltpu.async_copy(cat.at[0], out_slice(l0), sem_o0)
            pl.when(l0 + 2 < L)(lambda: gathers(l0 + 2, 0))
            wait_gathers(l0 + 1, 1)
            pl.when(g > 0)(lambda: wait_write(l0 + 1, 1, sem_o1))
            transpose_into_cat(1)
            pltpu.async_copy(cat.at[1], out_slice(l0 + 1), sem_o1)
            return carry

        lax.fori_loop(0, L // 2, step, 0)
        wait_write(L - 2, 0, sem_o0)
        wait_write(L - 1, 1, sem_o1)

    return k


_GATHER = _sc_gather_concat()


def kernel(word_inputs, feature_inputs_0, feature_inputs_1, word_seq_lengths,
           char_inputs, char_seq_lengths, char_seq_recover,
           word_table, feat_table_0, feat_table_1):
    out4 = _GATHER(word_inputs.T.astype(jnp.int32),
                   feature_inputs_0.T.astype(jnp.int32),
                   feature_inputs_1.T.astype(jnp.int32),
                   word_table, feat_table_0, feat_table_1)
    out5 = out4.reshape(L, CT, NW, 8, BT)
    return jnp.transpose(out5, (2, 4, 0, 1, 3)).reshape(B, L, OUT_D)
